# baseline (speedup 1.0000x reference)
import functools

import jax
import jax.numpy as jnp
from jax import lax
from jax.experimental import pallas as pl
from jax.experimental.pallas import tpu as pltpu
from jax.experimental.pallas import tpu_sc as plsc

VOC = 100000
EMB = 128
P = 16384
N = 65536

R = 16384
G = 7
S_PAD = G * R

NC = 2
NS = 16
NW = NC * NS
PP = P // NW
NN = N // NW
L = 16
KP = PP // 128
KN = NN // 128


def _tc_scores(lab_ref, vrow_ref, emb_ref, s_ref):
    dots = lax.dot_general(
        vrow_ref[0], emb_ref[...],
        (((1,), (1,)), ((), ())),
        preferred_element_type=jnp.float32,
    )
    s_ref[...] = (1.0 / (1.0 + jnp.exp(-dots))).reshape(R)


@functools.partial(
    pl.kernel,
    mesh=plsc.VectorSubcoreMesh(core_axis_name="c", subcore_axis_name="s"),
    out_type=(
        jax.ShapeDtypeStruct((NW, L), jnp.float32),
        jax.ShapeDtypeStruct((NW, L), jnp.float32),
    ),
    scratch_types=[
        pltpu.VMEM((PP,), jnp.int32),
        pltpu.VMEM((NN,), jnp.int32),
        pltpu.VMEM((PP,), jnp.float32),
        pltpu.VMEM((NN,), jnp.float32),
        pltpu.VMEM((L,), jnp.float32),
        pltpu.VMEM((L,), jnp.float32),
        pltpu.SemaphoreType.DMA,
    ],
)
def _sc_gather_sum(s_hbm, pos_hbm, neg_hbm, outp_hbm, outn_hbm,
                   idxp_v, idxn_v, valp_v, valn_v, accp_v, accn_v, sem):
    wid = lax.axis_index("s") * NC + lax.axis_index("c")
    pltpu.sync_copy(pos_hbm.at[pl.ds(wid * PP, PP)], idxp_v)
    pltpu.sync_copy(neg_hbm.at[pl.ds(wid * NN, NN)], idxn_v)

    copies = []
    for j in range(KP):
        copies.append(pltpu.async_copy(
            s_hbm.at[idxp_v.at[pl.ds(j * 128, 128)]],
            valp_v.at[pl.ds(j * 128, 128)], sem))
    for j in range(KN):
        copies.append(pltpu.async_copy(
            s_hbm.at[idxn_v.at[pl.ds(j * 128, 128)]],
            valn_v.at[pl.ds(j * 128, 128)], sem))
    for c in copies:
        c.wait()

    def body_p(i, acc):
        return acc + valp_v[pl.ds(pl.multiple_of(i * L, L), L)]

    accp_v[...] = lax.fori_loop(0, PP // L, body_p, jnp.zeros((L,), jnp.float32))
    pltpu.sync_copy(accp_v, outp_hbm.at[wid])

    def body_n(i, acc):
        return acc + valn_v[pl.ds(pl.multiple_of(i * L, L), L)]

    accn_v[...] = lax.fori_loop(0, NN // L, body_n, jnp.zeros((L,), jnp.float32))
    pltpu.sync_copy(accn_v, outn_hbm.at[wid])


def kernel(input_labels, pos_labels, neg_labels, in_embed, out_embed):
    s_flat = pl.pallas_call(
        _tc_scores,
        grid_spec=pltpu.PrefetchScalarGridSpec(
            num_scalar_prefetch=1,
            grid=(G,),
            in_specs=[
                pl.BlockSpec((1, 1, EMB), lambda i, lab: (lab[0], 0, 0)),
                pl.BlockSpec((R, EMB), lambda i, lab: (i, 0)),
            ],
            out_specs=pl.BlockSpec((R,), lambda i, lab: (i,)),
        ),
        out_shape=jax.ShapeDtypeStruct((S_PAD,), jnp.float32),
    )(input_labels, in_embed.reshape(VOC, 1, EMB), out_embed)

    part_p, part_n = _sc_gather_sum(s_flat, pos_labels, neg_labels)
    log_pos = jnp.sum(part_p).reshape(1, 1)
    log_neg = jnp.sum(part_n).reshape(1, 1)
    return (log_pos, log_neg)


# speedup vs baseline: 1.0732x; 1.0732x over previous
import functools

import jax
import jax.numpy as jnp
from jax import lax
from jax.experimental import pallas as pl
from jax.experimental.pallas import tpu as pltpu
from jax.experimental.pallas import tpu_sc as plsc

VOC = 100000
EMB = 128
P = 16384
N = 65536

R = 16384
G = 7
S_PAD = G * R

NC = 2
NS = 16
NW = NC * NS
PP = P // NW
NN = N // NW
L = 16
KP = PP // 128
KN = NN // 128


def _tc_scores(lab_ref, vrow_ref, emb_ref, s_ref):
    dots = lax.dot_general(
        vrow_ref[0], emb_ref[...],
        (((1,), (1,)), ((), ())),
        preferred_element_type=jnp.float32,
    )
    s_ref[...] = (1.0 / (1.0 + jnp.exp(-dots))).reshape(R)


SSEG = S_PAD // NS


@functools.partial(
    pl.kernel,
    mesh=plsc.VectorSubcoreMesh(core_axis_name="c", subcore_axis_name="s"),
    out_type=jax.ShapeDtypeStruct((NC, 2, L), jnp.float32),
    scratch_types=[
        pltpu.VMEM((PP,), jnp.int32),
        pltpu.VMEM((NN,), jnp.int32),
        pltpu.VMEM((PP,), jnp.float32),
        pltpu.VMEM((NN,), jnp.float32),
        pltpu.VMEM((2, L), jnp.float32),
        pltpu.VMEM((2, L), jnp.float32),
        pltpu.VMEM((16,), jnp.int32),
        pltpu.VMEM((16,), jnp.float32),
        pltpu.VMEM_SHARED((S_PAD,), jnp.float32),
        pltpu.VMEM_SHARED((2, L), jnp.float32),
        pltpu.SemaphoreType.DMA,
    ],
)
def _sc_gather_sum(s_hbm, pos_hbm, neg_hbm, out_hbm,
                   idxp_v, idxn_v, valp_v, valn_v, acc2_v, res2_v,
                   iota_v, out_v, s_sh, sacc_sh, sem):
    cid = lax.axis_index("c")
    sid = lax.axis_index("s")
    wid = sid * NC + cid
    pltpu.sync_copy(pos_hbm.at[pl.ds(wid * PP, PP)], idxp_v)
    pltpu.sync_copy(neg_hbm.at[pl.ds(wid * NN, NN)], idxn_v)
    pltpu.sync_copy(s_hbm.at[pl.ds(sid * SSEG, SSEG)],
                    s_sh.at[pl.ds(sid * SSEG, SSEG)])
    zero = jnp.zeros((L,), jnp.float32)

    @pl.when(sid == 0)
    def _zero_acc():
        acc2_v[0, :] = zero
        acc2_v[1, :] = zero
        pltpu.sync_copy(acc2_v, sacc_sh)

    plsc.subcore_barrier()

    copies = []
    for j in range(KP):
        copies.append(pltpu.async_copy(
            s_sh.at[idxp_v.at[pl.ds(j * 128, 128)]],
            valp_v.at[pl.ds(j * 128, 128)], sem))
    for j in range(KN):
        copies.append(pltpu.async_copy(
            s_sh.at[idxn_v.at[pl.ds(j * 128, 128)]],
            valn_v.at[pl.ds(j * 128, 128)], sem))
    for c in copies:
        c.wait()

    def body_p(i, acc):
        return acc + valp_v[pl.ds(pl.multiple_of(i * L, L), L)]

    accp = lax.fori_loop(0, PP // L, body_p, zero)

    def body_n(i, acc):
        return acc + valn_v[pl.ds(pl.multiple_of(i * L, L), L)]

    accn = lax.fori_loop(0, NN // L, body_n, zero)

    lane = lax.iota(jnp.int32, 16)
    acc2_v[0, :] = accp
    acc2_v[1, :] = accn
    iota_v[...] = lane
    pltpu.sync_copy(acc2_v, sacc_sh.at[iota_v.at[pl.ds(0, 2)]], add=True)
    plsc.subcore_barrier()

    @pl.when(sid == 0)
    def _finalize():
        pltpu.sync_copy(sacc_sh, res2_v)
        pltpu.sync_copy(res2_v, out_hbm.at[cid])


def kernel(input_labels, pos_labels, neg_labels, in_embed, out_embed):
    s_flat = pl.pallas_call(
        _tc_scores,
        grid_spec=pltpu.PrefetchScalarGridSpec(
            num_scalar_prefetch=1,
            grid=(G,),
            in_specs=[
                pl.BlockSpec((1, 1, EMB), lambda i, lab: (lab[0], 0, 0)),
                pl.BlockSpec((R, EMB), lambda i, lab: (i, 0)),
            ],
            out_specs=pl.BlockSpec((R,), lambda i, lab: (i,)),
        ),
        out_shape=jax.ShapeDtypeStruct((S_PAD,), jnp.float32),
    )(input_labels, in_embed.reshape(VOC, 1, EMB), out_embed)

    part = _sc_gather_sum(s_flat, pos_labels, neg_labels)
    log_pos = jnp.sum(part[:, 0, :]).reshape(1, 1)
    log_neg = jnp.sum(part[:, 1, :]).reshape(1, 1)
    return (log_pos, log_neg)
